# Initial kernel scaffold; baseline (speedup 1.0000x reference)
import jax
import jax.numpy as jnp
from jax.experimental import pallas as pl


def kernel(query_hidden, k_predicted, phase_idx, embeddings):
    raise NotImplementedError("write your pallas kernel here")


# speedup vs baseline: 1.8635x; 1.8635x over previous
import jax
import jax.numpy as jnp
from jax.experimental import pallas as pl
from jax.experimental.pallas import tpu as pltpu
from jax.experimental.pallas import tpu_sc as plsc

POOL = 100000
POOL_PAD = 100352
DIM = 256
K = 64
QB = 256
PB = 2048
NQ = 2048
NEG = float("-inf")


def _chunk_mask(c, lim):
    it = jax.lax.broadcasted_iota(jnp.int32, (1, 1, c, 1), 2)
    return it < lim


def _cmpex(v, ix, stride, desc_mask):
    s = v.shape[0]
    g = s // (2 * stride)
    v5 = v.reshape(g, 2, stride, *v.shape[1:])
    i5 = ix.reshape(g, 2, stride, *ix.shape[1:])
    av, bv = v5[:, 0], v5[:, 1]
    ai, bi = i5[:, 0], i5[:, 1]
    gt = (av > bv) | ((av == bv) & (ai < bi))
    take = gt == desc_mask
    nav = jnp.where(take, av, bv)
    nbv = jnp.where(take, bv, av)
    nai = jnp.where(take, ai, bi)
    nbi = jnp.where(take, bi, ai)
    v = jnp.stack([nav, nbv], axis=1).reshape(*v.shape)
    ix = jnp.stack([nai, nbi], axis=1).reshape(*ix.shape)
    return v, ix


def _sort64(v, ix, chunk_desc):
    for k in (2, 4, 8, 16, 32, 64):
        stride = k // 2
        while stride >= 1:
            g = 64 // (2 * stride)
            m = k // (2 * stride)
            giota = jax.lax.broadcasted_iota(jnp.int32, (g, 1, 1, 1), 0)
            stage_desc = ((giota // m) % 2) == 0
            v, ix = _cmpex(v, ix, stride, stage_desc == chunk_desc)
            stride //= 2
    return v, ix


def _merge_level(v, ix, out_desc_mask):
    c = v.shape[1]
    av, ai = v[:, : c // 2], ix[:, : c // 2]
    bv, bi = v[:, c // 2 :], ix[:, c // 2 :]
    gt = (av > bv) | ((av == bv) & (ai < bi))
    nv = jnp.where(gt, av, bv)
    ni = jnp.where(gt, ai, bi)
    for stride in (32, 16, 8, 4, 2, 1):
        nv, ni = _cmpex(nv, ni, stride, out_desc_mask)
    return nv, ni


def _block_top64(s, rid, final_desc):
    c = s.shape[1]
    v, ix = _sort64(s, rid, _chunk_mask(c, c // 2))
    while c > 1:
        m = c // 2
        mask = final_desc if m == 1 else _chunk_mask(m, m // 2)
        v, ix = _merge_level(v, ix, mask)
        c = m
    return v, ix


def _topk_kernel(q_ref, e_ref, k_ref, o_ref, vals_scr, idx_scr):
    j = pl.program_id(1)
    nj = pl.num_programs(1)

    e = e_ref[...]
    q = q_ref[...]
    s = jax.lax.dot_general(e, q, (((1,), (1,)), ((), ())),
                            preferred_element_type=jnp.float32)

    s3 = s.reshape(64, PB // 64, QB)
    slot = jax.lax.broadcasted_iota(jnp.int32, (64, PB // 64, QB), 0)
    chunk = jax.lax.broadcasted_iota(jnp.int32, (64, PB // 64, QB), 1)
    rid = j * PB + slot * (PB // 64) + chunk
    s3 = jnp.where(rid < POOL, s3, NEG)

    bv, bi = _block_top64(s3, rid, final_desc=False)

    @pl.when(j == 0)
    def _():
        vals_scr[...] = jnp.full((64, QB), NEG, jnp.float32)
        idx_scr[...] = jnp.zeros((64, QB), jnp.int32)

    rv = jnp.stack([vals_scr[...], bv[:, 0]], axis=1)
    ri = jnp.stack([idx_scr[...], bi[:, 0]], axis=1)
    mv, mi = _merge_level(rv, ri, True)
    vals_scr[...] = mv[:, 0]
    idx_scr[...] = mi[:, 0]

    @pl.when(j == nj - 1)
    def _():
        kv = k_ref[0, 0, :]
        slot_q = jax.lax.broadcasted_iota(jnp.int32, (64, QB), 0)
        keep = slot_q < kv[None, :]
        o_ref[0] = jnp.where(keep, mi[:, 0], POOL)


def _run_topk(flat_q_t, emb_pad, k3):
    return pl.pallas_call(
        _topk_kernel,
        grid=(NQ // QB, POOL_PAD // PB),
        in_specs=[
            pl.BlockSpec((QB, DIM), lambda i, j: (i, 0)),
            pl.BlockSpec((PB, DIM), lambda i, j: (j, 0)),
            pl.BlockSpec((1, 1, QB), lambda i, j: (i, 0, 0)),
        ],
        out_specs=pl.BlockSpec((1, 64, QB), lambda i, j: (i, 0, 0)),
        out_shape=jax.ShapeDtypeStruct((NQ // QB, 64, QB), jnp.int32),
        scratch_shapes=[
            pltpu.VMEM((64, QB), jnp.float32),
            pltpu.VMEM((64, QB), jnp.int32),
        ],
        compiler_params=pltpu.CompilerParams(
            dimension_semantics=("parallel", "arbitrary")),
    )(flat_q_t, emb_pad, k3)


def _gather_rows(emb_pad, idx_flat):
    n = idx_flat.shape[1]
    w = 128
    half = n // 2
    mesh = plsc.VectorSubcoreMesh(core_axis_name="core", subcore_axis_name="subcore")

    @pl.kernel(out_type=jax.ShapeDtypeStruct((n, DIM), jnp.float32), mesh=mesh)
    def gk(x_hbm, i_hbm, o_hbm):
        c = jax.lax.axis_index("core")

        def body(i_vmem, o_vmem):
            pltpu.sync_copy(x_hbm.at[i_vmem.at[0]], o_vmem)

        pltpu.emit_pipeline(
            body,
            grid=(half // w,),
            in_specs=[pl.BlockSpec((1, w), index_map=lambda i: (0, i))],
            out_specs=[pl.BlockSpec((w, DIM), index_map=lambda i: (i, 0))],
            core_axis_name="subcore",
            dimension_semantics=(pltpu.PARALLEL,),
        )(i_hbm.at[:, pl.ds(c * half, half)], o_hbm.at[pl.ds(c * half, half)])

    return gk(emb_pad, idx_flat)


def kernel(query_hidden, k_predicted, phase_idx, embeddings):
    batch, seq, dim = query_hidden.shape
    flat_q = query_hidden.reshape(-1, dim)
    emb_pad = jnp.pad(embeddings, ((0, POOL_PAD - POOL), (0, 0)))
    k3 = k_predicted.reshape(NQ // QB, 1, QB)

    top_idx = _run_topk(flat_q, emb_pad, k3)
    idx_flat = top_idx.transpose(0, 2, 1).reshape(1, NQ * K)

    rows = _gather_rows(emb_pad, idx_flat)
    return rows.reshape(batch, seq, K, dim)


# speedup vs baseline: 2.1190x; 1.1371x over previous
import jax
import jax.numpy as jnp
from jax.experimental import pallas as pl
from jax.experimental.pallas import tpu as pltpu
from jax.experimental.pallas import tpu_sc as plsc

POOL = 100000
POOL_PAD = 100352
DIM = 256
K = 64
QB = 256
PB = 2048
NQ = 2048
NEG = float("-inf")


def _chunk_mask(c, lim):
    it = jax.lax.broadcasted_iota(jnp.int32, (1, 1, c, 1), 2)
    return it < lim


def _cmpex(v, ix, stride, desc_mask):
    s = v.shape[0]
    g = s // (2 * stride)
    v5 = v.reshape(g, 2, stride, *v.shape[1:])
    i5 = ix.reshape(g, 2, stride, *ix.shape[1:])
    av, bv = v5[:, 0], v5[:, 1]
    ai, bi = i5[:, 0], i5[:, 1]
    gt = (av > bv) | ((av == bv) & (ai < bi))
    take = gt == desc_mask
    nav = jnp.where(take, av, bv)
    nbv = jnp.where(take, bv, av)
    nai = jnp.where(take, ai, bi)
    nbi = jnp.where(take, bi, ai)
    v = jnp.stack([nav, nbv], axis=1).reshape(*v.shape)
    ix = jnp.stack([nai, nbi], axis=1).reshape(*ix.shape)
    return v, ix


def _sort64(v, ix, chunk_desc):
    for k in (2, 4, 8, 16, 32, 64):
        stride = k // 2
        while stride >= 1:
            g = 64 // (2 * stride)
            m = k // (2 * stride)
            giota = jax.lax.broadcasted_iota(jnp.int32, (g, 1, 1, 1), 0)
            stage_desc = ((giota // m) % 2) == 0
            v, ix = _cmpex(v, ix, stride, stage_desc == chunk_desc)
            stride //= 2
    return v, ix


def _merge_level(v, ix, out_desc_mask):
    c = v.shape[1]
    av, ai = v[:, : c // 2], ix[:, : c // 2]
    bv, bi = v[:, c // 2 :], ix[:, c // 2 :]
    gt = (av > bv) | ((av == bv) & (ai < bi))
    nv = jnp.where(gt, av, bv)
    ni = jnp.where(gt, ai, bi)
    for stride in (32, 16, 8, 4, 2, 1):
        nv, ni = _cmpex(nv, ni, stride, out_desc_mask)
    return nv, ni


def _block_top64(s, rid, final_desc):
    c = s.shape[1]
    v, ix = _sort64(s, rid, _chunk_mask(c, c // 2))
    while c > 1:
        m = c // 2
        mask = final_desc if m == 1 else _chunk_mask(m, m // 2)
        v, ix = _merge_level(v, ix, mask)
        c = m
    return v, ix


def _topk_kernel(q_ref, e_ref, k_ref, o_ref, vals_scr, idx_scr):
    j = pl.program_id(1)
    nj = pl.num_programs(1)

    e = e_ref[...]
    q = q_ref[...]
    s = jax.lax.dot_general(e, q, (((1,), (1,)), ((), ())),
                            preferred_element_type=jnp.float32)

    s3 = s.reshape(64, PB // 64, QB)
    slot = jax.lax.broadcasted_iota(jnp.int32, (64, PB // 64, QB), 0)
    chunk = jax.lax.broadcasted_iota(jnp.int32, (64, PB // 64, QB), 1)
    rid = j * PB + slot * (PB // 64) + chunk
    s3 = jnp.where(rid < POOL, s3, NEG)

    bv, bi = _block_top64(s3, rid, final_desc=False)

    @pl.when(j == 0)
    def _():
        vals_scr[...] = jnp.full((64, QB), NEG, jnp.float32)
        idx_scr[...] = jnp.zeros((64, QB), jnp.int32)

    rv = jnp.stack([vals_scr[...], bv[:, 0]], axis=1)
    ri = jnp.stack([idx_scr[...], bi[:, 0]], axis=1)
    mv, mi = _merge_level(rv, ri, True)
    vals_scr[...] = mv[:, 0]
    idx_scr[...] = mi[:, 0]

    @pl.when(j == nj - 1)
    def _():
        kv = k_ref[0, 0, :]
        slot_q = jax.lax.broadcasted_iota(jnp.int32, (64, QB), 0)
        keep = slot_q < kv[None, :]
        o_ref[0] = jnp.where(keep, mi[:, 0], POOL)


def _run_topk(flat_q_t, emb_pad, k3):
    nqb = flat_q_t.shape[0] // QB
    return pl.pallas_call(
        _topk_kernel,
        grid=(nqb, POOL_PAD // PB),
        in_specs=[
            pl.BlockSpec((QB, DIM), lambda i, j: (i, 0)),
            pl.BlockSpec((PB, DIM), lambda i, j: (j, 0)),
            pl.BlockSpec((1, 1, QB), lambda i, j: (i, 0, 0)),
        ],
        out_specs=pl.BlockSpec((1, 64, QB), lambda i, j: (i, 0, 0)),
        out_shape=jax.ShapeDtypeStruct((nqb, 64, QB), jnp.int32),
        scratch_shapes=[
            pltpu.VMEM((64, QB), jnp.float32),
            pltpu.VMEM((64, QB), jnp.int32),
        ],
        compiler_params=pltpu.CompilerParams(
            dimension_semantics=("parallel", "arbitrary")),
    )(flat_q_t, emb_pad, k3)


def _gather_rows(emb_pad, idx_flat):
    n = idx_flat.shape[1]
    w = 128
    half = n // 2
    mesh = plsc.VectorSubcoreMesh(core_axis_name="core", subcore_axis_name="subcore")

    @pl.kernel(out_type=jax.ShapeDtypeStruct((n, DIM), jnp.float32), mesh=mesh)
    def gk(x_hbm, i_hbm, o_hbm):
        c = jax.lax.axis_index("core")

        def body(i_vmem, o_vmem):
            pltpu.sync_copy(x_hbm.at[i_vmem.at[0]], o_vmem)

        pltpu.emit_pipeline(
            body,
            grid=(half // w,),
            in_specs=[pl.BlockSpec((1, w), index_map=lambda i: (0, i))],
            out_specs=[pl.BlockSpec((w, DIM), index_map=lambda i: (i, 0))],
            core_axis_name="subcore",
            dimension_semantics=(pltpu.PARALLEL,),
        )(i_hbm.at[:, pl.ds(c * half, half)], o_hbm.at[pl.ds(c * half, half)])

    return gk(emb_pad, idx_flat)


def kernel(query_hidden, k_predicted, phase_idx, embeddings):
    batch, seq, dim = query_hidden.shape
    flat_q = query_hidden.reshape(-1, dim)
    emb_pad = jnp.pad(embeddings, ((0, POOL_PAD - POOL), (0, 0)))
    k3 = k_predicted.reshape(NQ // QB, 1, QB)

    nsplit = 4
    qb_per = NQ // nsplit
    rows = []
    for s in range(nsplit):
        top_idx = _run_topk(
            jax.lax.dynamic_slice_in_dim(flat_q, s * qb_per, qb_per, 0),
            emb_pad,
            jax.lax.dynamic_slice_in_dim(k3, s * (qb_per // QB), qb_per // QB, 0),
        )
        idx_flat = top_idx.transpose(0, 2, 1).reshape(1, qb_per * K)
        rows.append(_gather_rows(emb_pad, idx_flat))

    out = jnp.concatenate(rows, axis=0)
    return out.reshape(batch, seq, K, dim)


# speedup vs baseline: 2.1605x; 1.0196x over previous
import jax
import jax.numpy as jnp
from jax.experimental import pallas as pl
from jax.experimental.pallas import tpu as pltpu
from jax.experimental.pallas import tpu_sc as plsc

POOL = 100000
POOL_PAD = 100352
DIM = 256
K = 64
QB = 256
PB = 2048
NQ = 2048
NEG = float("-inf")


def _chunk_mask(c, lim):
    it = jax.lax.broadcasted_iota(jnp.int32, (1, 1, c, 1), 2)
    return it < lim


def _cmpex(v, ix, stride, desc_mask):
    s = v.shape[0]
    g = s // (2 * stride)
    v5 = v.reshape(g, 2, stride, *v.shape[1:])
    i5 = ix.reshape(g, 2, stride, *ix.shape[1:])
    av, bv = v5[:, 0], v5[:, 1]
    ai, bi = i5[:, 0], i5[:, 1]
    gt = (av > bv) | ((av == bv) & (ai < bi))
    take = gt == desc_mask
    nav = jnp.where(take, av, bv)
    nbv = jnp.where(take, bv, av)
    nai = jnp.where(take, ai, bi)
    nbi = jnp.where(take, bi, ai)
    v = jnp.stack([nav, nbv], axis=1).reshape(*v.shape)
    ix = jnp.stack([nai, nbi], axis=1).reshape(*ix.shape)
    return v, ix


def _sort64(v, ix, chunk_desc):
    for k in (2, 4, 8, 16, 32, 64):
        stride = k // 2
        while stride >= 1:
            g = 64 // (2 * stride)
            m = k // (2 * stride)
            giota = jax.lax.broadcasted_iota(jnp.int32, (g, 1, 1, 1), 0)
            stage_desc = ((giota // m) % 2) == 0
            v, ix = _cmpex(v, ix, stride, stage_desc == chunk_desc)
            stride //= 2
    return v, ix


def _merge_level(v, ix, out_desc_mask):
    c = v.shape[1]
    av, ai = v[:, : c // 2], ix[:, : c // 2]
    bv, bi = v[:, c // 2 :], ix[:, c // 2 :]
    gt = (av > bv) | ((av == bv) & (ai < bi))
    nv = jnp.where(gt, av, bv)
    ni = jnp.where(gt, ai, bi)
    for stride in (32, 16, 8, 4, 2, 1):
        nv, ni = _cmpex(nv, ni, stride, out_desc_mask)
    return nv, ni


def _block_top64(s, rid, final_desc):
    c = s.shape[1]
    v, ix = _sort64(s, rid, _chunk_mask(c, c // 2))
    while c > 1:
        m = c // 2
        mask = final_desc if m == 1 else _chunk_mask(m, m // 2)
        v, ix = _merge_level(v, ix, mask)
        c = m
    return v, ix


def _topk_kernel(q_ref, e_ref, k_ref, o_ref, vals_scr, idx_scr):
    j = pl.program_id(1)
    nj = pl.num_programs(1)

    e = e_ref[...]
    q = q_ref[...]
    s = jax.lax.dot_general(e, q, (((1,), (1,)), ((), ())),
                            preferred_element_type=jnp.float32)

    s3 = s.reshape(64, PB // 64, QB)
    slot = jax.lax.broadcasted_iota(jnp.int32, (64, PB // 64, QB), 0)
    chunk = jax.lax.broadcasted_iota(jnp.int32, (64, PB // 64, QB), 1)
    rid = j * PB + slot * (PB // 64) + chunk
    s3 = jnp.where(rid < POOL, s3, NEG)

    bv, bi = _block_top64(s3, rid, final_desc=False)

    @pl.when(j == 0)
    def _():
        vals_scr[...] = jnp.full((64, QB), NEG, jnp.float32)
        idx_scr[...] = jnp.zeros((64, QB), jnp.int32)

    rv = jnp.stack([vals_scr[...], bv[:, 0]], axis=1)
    ri = jnp.stack([idx_scr[...], bi[:, 0]], axis=1)
    mv, mi = _merge_level(rv, ri, True)
    vals_scr[...] = mv[:, 0]
    idx_scr[...] = mi[:, 0]

    @pl.when(j == nj - 1)
    def _():
        kv = k_ref[0, 0, :]
        slot_q = jax.lax.broadcasted_iota(jnp.int32, (64, QB), 0)
        keep = slot_q < kv[None, :]
        o_ref[0] = jnp.where(keep, mi[:, 0], POOL)


def _run_topk(flat_q_t, emb_pad, k3):
    nqb = flat_q_t.shape[0] // QB
    return pl.pallas_call(
        _topk_kernel,
        grid=(nqb, POOL_PAD // PB),
        in_specs=[
            pl.BlockSpec((QB, DIM), lambda i, j: (i, 0)),
            pl.BlockSpec((PB, DIM), lambda i, j: (j, 0)),
            pl.BlockSpec((1, 1, QB), lambda i, j: (i, 0, 0)),
        ],
        out_specs=pl.BlockSpec((1, 64, QB), lambda i, j: (i, 0, 0)),
        out_shape=jax.ShapeDtypeStruct((nqb, 64, QB), jnp.int32),
        scratch_shapes=[
            pltpu.VMEM((64, QB), jnp.float32),
            pltpu.VMEM((64, QB), jnp.int32),
        ],
        compiler_params=pltpu.CompilerParams(
            dimension_semantics=("parallel", "arbitrary")),
    )(flat_q_t, emb_pad, k3)


def _gather_rows(emb_pad, idx_flat):
    n = idx_flat.shape[1]
    w = 128
    half = n // 2
    mesh = plsc.VectorSubcoreMesh(core_axis_name="core", subcore_axis_name="subcore")

    @pl.kernel(out_type=jax.ShapeDtypeStruct((n, DIM), jnp.float32), mesh=mesh)
    def gk(x_hbm, i_hbm, o_hbm):
        c = jax.lax.axis_index("core")

        def body(i_vmem, o_vmem):
            pltpu.sync_copy(x_hbm.at[i_vmem.at[0]], o_vmem)

        pltpu.emit_pipeline(
            body,
            grid=(half // w,),
            in_specs=[pl.BlockSpec((1, w), index_map=lambda i: (0, i))],
            out_specs=[pl.BlockSpec((w, DIM), index_map=lambda i: (i, 0))],
            core_axis_name="subcore",
            dimension_semantics=(pltpu.PARALLEL,),
        )(i_hbm.at[:, pl.ds(c * half, half)], o_hbm.at[pl.ds(c * half, half)])

    return gk(emb_pad, idx_flat)


def kernel(query_hidden, k_predicted, phase_idx, embeddings):
    batch, seq, dim = query_hidden.shape
    flat_q = query_hidden.reshape(-1, dim)
    emb_pad = jnp.pad(embeddings, ((0, POOL_PAD - POOL), (0, 0)))
    k3 = k_predicted.reshape(NQ // QB, 1, QB)

    nsplit = 8
    qb_per = NQ // nsplit
    rows = []
    for s in range(nsplit):
        top_idx = _run_topk(
            jax.lax.dynamic_slice_in_dim(flat_q, s * qb_per, qb_per, 0),
            emb_pad,
            jax.lax.dynamic_slice_in_dim(k3, s * (qb_per // QB), qb_per // QB, 0),
        )
        idx_flat = top_idx.transpose(0, 2, 1).reshape(1, qb_per * K)
        rows.append(_gather_rows(emb_pad, idx_flat))

    out = jnp.concatenate(rows, axis=0)
    return out.reshape(batch, seq, K, dim)


# speedup vs baseline: 2.1659x; 1.0025x over previous
import jax
import jax.numpy as jnp
from jax.experimental import pallas as pl
from jax.experimental.pallas import tpu as pltpu
from jax.experimental.pallas import tpu_sc as plsc

POOL = 100000
POOL_PAD = 100352
DIM = 256
K = 64
QB = 256
PB = 2048
NQ = 2048
NEG = float("-inf")


def _chunk_mask(c, lim):
    it = jax.lax.broadcasted_iota(jnp.int32, (1, 1, c, 1), 2)
    return it < lim


def _cmpex(v, ix, stride, desc_mask):
    s = v.shape[0]
    g = s // (2 * stride)
    v5 = v.reshape(g, 2, stride, *v.shape[1:])
    i5 = ix.reshape(g, 2, stride, *ix.shape[1:])
    av, bv = v5[:, 0], v5[:, 1]
    ai, bi = i5[:, 0], i5[:, 1]
    gt = (av > bv) | ((av == bv) & (ai < bi))
    take = gt == desc_mask
    nav = jnp.where(take, av, bv)
    nbv = jnp.where(take, bv, av)
    nai = jnp.where(take, ai, bi)
    nbi = jnp.where(take, bi, ai)
    v = jnp.stack([nav, nbv], axis=1).reshape(*v.shape)
    ix = jnp.stack([nai, nbi], axis=1).reshape(*ix.shape)
    return v, ix


def _sort64(v, ix, chunk_desc):
    for k in (2, 4, 8, 16, 32, 64):
        stride = k // 2
        while stride >= 1:
            g = 64 // (2 * stride)
            m = k // (2 * stride)
            giota = jax.lax.broadcasted_iota(jnp.int32, (g, 1, 1, 1), 0)
            stage_desc = ((giota // m) % 2) == 0
            v, ix = _cmpex(v, ix, stride, stage_desc == chunk_desc)
            stride //= 2
    return v, ix


def _merge_level(v, ix, out_desc_mask):
    c = v.shape[1]
    av, ai = v[:, : c // 2], ix[:, : c // 2]
    bv, bi = v[:, c // 2 :], ix[:, c // 2 :]
    gt = (av > bv) | ((av == bv) & (ai < bi))
    nv = jnp.where(gt, av, bv)
    ni = jnp.where(gt, ai, bi)
    for stride in (32, 16, 8, 4, 2, 1):
        nv, ni = _cmpex(nv, ni, stride, out_desc_mask)
    return nv, ni


def _block_top64(s, rid, final_desc):
    c = s.shape[1]
    v, ix = _sort64(s, rid, _chunk_mask(c, c // 2))
    while c > 1:
        m = c // 2
        mask = final_desc if m == 1 else _chunk_mask(m, m // 2)
        v, ix = _merge_level(v, ix, mask)
        c = m
    return v, ix


def _topk_kernel(q_ref, e_ref, k_ref, o_ref, sc_scr, vals_scr, idx_scr):
    j = pl.program_id(1)
    nj = pl.num_programs(1)

    e = e_ref[...]
    q = q_ref[...]
    s = jax.lax.dot_general(e, q, (((1,), (1,)), ((), ())),
                            preferred_element_type=jnp.float32)
    sc_scr[j % 2] = s

    jj = j - 1
    sp = sc_scr[(j + 1) % 2]
    s3 = sp.reshape(64, PB // 64, QB)
    slot = jax.lax.broadcasted_iota(jnp.int32, (64, PB // 64, QB), 0)
    chunk = jax.lax.broadcasted_iota(jnp.int32, (64, PB // 64, QB), 1)
    rid = jj * PB + slot * (PB // 64) + chunk
    s3 = jnp.where((rid < POOL) & (rid >= 0), s3, NEG)

    bv, bi = _block_top64(s3, rid, final_desc=False)

    @pl.when(j == 1)
    def _():
        vals_scr[...] = jnp.full((64, QB), NEG, jnp.float32)
        idx_scr[...] = jnp.zeros((64, QB), jnp.int32)

    rv = jnp.stack([vals_scr[...], bv[:, 0]], axis=1)
    ri = jnp.stack([idx_scr[...], bi[:, 0]], axis=1)
    mv, mi = _merge_level(rv, ri, True)
    vals_scr[...] = mv[:, 0]
    idx_scr[...] = mi[:, 0]

    @pl.when(j == nj - 1)
    def _():
        kv = k_ref[0, 0, :]
        slot_q = jax.lax.broadcasted_iota(jnp.int32, (64, QB), 0)
        keep = slot_q < kv[None, :]
        o_ref[0] = jnp.where(keep, mi[:, 0], POOL)


def _run_topk(flat_q_t, emb_pad, k3):
    nqb = flat_q_t.shape[0] // QB
    npb = POOL_PAD // PB
    return pl.pallas_call(
        _topk_kernel,
        grid=(nqb, npb + 1),
        in_specs=[
            pl.BlockSpec((QB, DIM), lambda i, j: (i, 0)),
            pl.BlockSpec((PB, DIM), lambda i, j: (jnp.minimum(j, npb - 1), 0)),
            pl.BlockSpec((1, 1, QB), lambda i, j: (i, 0, 0)),
        ],
        out_specs=pl.BlockSpec((1, 64, QB), lambda i, j: (i, 0, 0)),
        out_shape=jax.ShapeDtypeStruct((nqb, 64, QB), jnp.int32),
        scratch_shapes=[
            pltpu.VMEM((2, PB, QB), jnp.float32),
            pltpu.VMEM((64, QB), jnp.float32),
            pltpu.VMEM((64, QB), jnp.int32),
        ],
        compiler_params=pltpu.CompilerParams(
            dimension_semantics=("parallel", "arbitrary")),
    )(flat_q_t, emb_pad, k3)


def _gather_rows(emb_pad, idx_flat):
    n = idx_flat.shape[1]
    w = 128
    half = n // 2
    mesh = plsc.VectorSubcoreMesh(core_axis_name="core", subcore_axis_name="subcore")

    @pl.kernel(out_type=jax.ShapeDtypeStruct((n, DIM), jnp.float32), mesh=mesh)
    def gk(x_hbm, i_hbm, o_hbm):
        c = jax.lax.axis_index("core")

        def body(i_vmem, o_vmem):
            pltpu.sync_copy(x_hbm.at[i_vmem.at[0]], o_vmem)

        pltpu.emit_pipeline(
            body,
            grid=(half // w,),
            in_specs=[pl.BlockSpec((1, w), index_map=lambda i: (0, i))],
            out_specs=[pl.BlockSpec((w, DIM), index_map=lambda i: (i, 0))],
            core_axis_name="subcore",
            dimension_semantics=(pltpu.PARALLEL,),
        )(i_hbm.at[:, pl.ds(c * half, half)], o_hbm.at[pl.ds(c * half, half)])

    return gk(emb_pad, idx_flat)


def kernel(query_hidden, k_predicted, phase_idx, embeddings):
    batch, seq, dim = query_hidden.shape
    flat_q = query_hidden.reshape(-1, dim)
    emb_pad = jnp.pad(embeddings, ((0, POOL_PAD - POOL), (0, 0)))
    k3 = k_predicted.reshape(NQ // QB, 1, QB)

    nsplit = 8
    qb_per = NQ // nsplit
    rows = []
    for s in range(nsplit):
        top_idx = _run_topk(
            jax.lax.dynamic_slice_in_dim(flat_q, s * qb_per, qb_per, 0),
            emb_pad,
            jax.lax.dynamic_slice_in_dim(k3, s * (qb_per // QB), qb_per // QB, 0),
        )
        idx_flat = top_idx.transpose(0, 2, 1).reshape(1, qb_per * K)
        rows.append(_gather_rows(emb_pad, idx_flat))

    out = jnp.concatenate(rows, axis=0)
    return out.reshape(batch, seq, K, dim)


# speedup vs baseline: 2.4244x; 1.1193x over previous
import jax
import jax.numpy as jnp
from jax.experimental import pallas as pl
from jax.experimental.pallas import tpu as pltpu
from jax.experimental.pallas import tpu_sc as plsc

POOL = 100000
POOL_PAD = 102400
DIM = 256
K = 64
QB = 256
PB = 2048
NQ = 2048
NEG = float("-inf")


def _chunk_mask(c, lim):
    it = jax.lax.broadcasted_iota(jnp.int32, (1, 1, c, 1), 2)
    return it < lim


def _cmpex(v, ix, stride, desc_mask):
    s = v.shape[0]
    g = s // (2 * stride)
    v5 = v.reshape(g, 2, stride, *v.shape[1:])
    i5 = ix.reshape(g, 2, stride, *ix.shape[1:])
    av, bv = v5[:, 0], v5[:, 1]
    ai, bi = i5[:, 0], i5[:, 1]
    gt = (av > bv) | ((av == bv) & (ai < bi))
    take = gt == desc_mask
    nav = jnp.where(take, av, bv)
    nbv = jnp.where(take, bv, av)
    nai = jnp.where(take, ai, bi)
    nbi = jnp.where(take, bi, ai)
    v = jnp.stack([nav, nbv], axis=1).reshape(*v.shape)
    ix = jnp.stack([nai, nbi], axis=1).reshape(*ix.shape)
    return v, ix


def _sort64(v, ix, chunk_desc):
    for k in (2, 4, 8, 16, 32, 64):
        stride = k // 2
        while stride >= 1:
            g = 64 // (2 * stride)
            m = k // (2 * stride)
            giota = jax.lax.broadcasted_iota(jnp.int32, (g, 1, 1, 1), 0)
            stage_desc = ((giota // m) % 2) == 0
            v, ix = _cmpex(v, ix, stride, stage_desc == chunk_desc)
            stride //= 2
    return v, ix


def _merge_level(v, ix, out_desc_mask):
    c = v.shape[1]
    av, ai = v[:, : c // 2], ix[:, : c // 2]
    bv, bi = v[:, c // 2 :], ix[:, c // 2 :]
    gt = (av > bv) | ((av == bv) & (ai < bi))
    nv = jnp.where(gt, av, bv)
    ni = jnp.where(gt, ai, bi)
    for stride in (32, 16, 8, 4, 2, 1):
        nv, ni = _cmpex(nv, ni, stride, out_desc_mask)
    return nv, ni


def _block_top64(s, rid, final_desc):
    c = s.shape[1]
    v, ix = _sort64(s, rid, _chunk_mask(c, c // 2))
    while c > 1:
        m = c // 2
        mask = final_desc if m == 1 else _chunk_mask(m, m // 2)
        v, ix = _merge_level(v, ix, mask)
        c = m
    return v, ix


def _block_sorted_top64(s, base_rid):
    s3 = s.reshape(64, PB // 64, QB)
    slot = jax.lax.broadcasted_iota(jnp.int32, (64, PB // 64, QB), 0)
    chunk = jax.lax.broadcasted_iota(jnp.int32, (64, PB // 64, QB), 1)
    rid = base_rid + slot * (PB // 64) + chunk
    s3 = jnp.where(rid < POOL, s3, NEG)
    return _block_top64(s3, rid, final_desc=False)


def _topk_kernel(q_ref, e_ref, k_ref, o_ref, vals_scr, idx_scr):
    j = pl.program_id(1)
    nj = pl.num_programs(1)

    e = e_ref[...]
    q = q_ref[...]
    s_a = jax.lax.dot_general(e[:PB], q, (((1,), (1,)), ((), ())),
                              preferred_element_type=jnp.float32)
    av, ai = _block_sorted_top64(s_a, 2 * j * PB)
    s_b = jax.lax.dot_general(e[PB:], q, (((1,), (1,)), ((), ())),
                              preferred_element_type=jnp.float32)

    @pl.when(j == 0)
    def _():
        vals_scr[...] = jnp.full((64, QB), NEG, jnp.float32)
        idx_scr[...] = jnp.zeros((64, QB), jnp.int32)

    rv = jnp.stack([vals_scr[...], av[:, 0]], axis=1)
    ri = jnp.stack([idx_scr[...], ai[:, 0]], axis=1)
    mv, mi = _merge_level(rv, ri, True)

    bv, bi = _block_sorted_top64(s_b, (2 * j + 1) * PB)
    rv = jnp.stack([mv[:, 0], bv[:, 0]], axis=1)
    ri = jnp.stack([mi[:, 0], bi[:, 0]], axis=1)
    mv, mi = _merge_level(rv, ri, True)
    vals_scr[...] = mv[:, 0]
    idx_scr[...] = mi[:, 0]

    @pl.when(j == nj - 1)
    def _():
        kv = k_ref[0, 0, :]
        slot_q = jax.lax.broadcasted_iota(jnp.int32, (64, QB), 0)
        keep = slot_q < kv[None, :]
        o_ref[0] = jnp.where(keep, mi[:, 0], POOL)


def _run_topk(flat_q_t, emb_pad, k3):
    nqb = flat_q_t.shape[0] // QB
    return pl.pallas_call(
        _topk_kernel,
        grid=(nqb, POOL_PAD // (2 * PB)),
        in_specs=[
            pl.BlockSpec((QB, DIM), lambda i, j: (i, 0)),
            pl.BlockSpec((2 * PB, DIM), lambda i, j: (j, 0)),
            pl.BlockSpec((1, 1, QB), lambda i, j: (i, 0, 0)),
        ],
        out_specs=pl.BlockSpec((1, 64, QB), lambda i, j: (i, 0, 0)),
        out_shape=jax.ShapeDtypeStruct((nqb, 64, QB), jnp.int32),
        scratch_shapes=[
            pltpu.VMEM((64, QB), jnp.float32),
            pltpu.VMEM((64, QB), jnp.int32),
        ],
        compiler_params=pltpu.CompilerParams(
            dimension_semantics=("parallel", "arbitrary")),
    )(flat_q_t, emb_pad, k3)


def _gather_rows(emb_pad, idx_flat):
    n = idx_flat.shape[1]
    w = 128
    half = n // 2
    mesh = plsc.VectorSubcoreMesh(core_axis_name="core", subcore_axis_name="subcore")

    @pl.kernel(out_type=jax.ShapeDtypeStruct((n, DIM), jnp.float32), mesh=mesh)
    def gk(x_hbm, i_hbm, o_hbm):
        c = jax.lax.axis_index("core")

        def body(i_vmem, o_vmem):
            pltpu.sync_copy(x_hbm.at[i_vmem.at[0]], o_vmem)

        pltpu.emit_pipeline(
            body,
            grid=(half // w,),
            in_specs=[pl.BlockSpec((1, w), index_map=lambda i: (0, i))],
            out_specs=[pl.BlockSpec((w, DIM), index_map=lambda i: (i, 0))],
            core_axis_name="subcore",
            dimension_semantics=(pltpu.PARALLEL,),
        )(i_hbm.at[:, pl.ds(c * half, half)], o_hbm.at[pl.ds(c * half, half)])

    return gk(emb_pad, idx_flat)


def kernel(query_hidden, k_predicted, phase_idx, embeddings):
    batch, seq, dim = query_hidden.shape
    flat_q = query_hidden.reshape(-1, dim)
    emb_pad = jnp.pad(embeddings, ((0, POOL_PAD - POOL), (0, 0)))
    k3 = k_predicted.reshape(NQ // QB, 1, QB)

    nsplit = 8
    qb_per = NQ // nsplit
    rows = []
    for s in range(nsplit):
        top_idx = _run_topk(
            jax.lax.dynamic_slice_in_dim(flat_q, s * qb_per, qb_per, 0),
            emb_pad,
            jax.lax.dynamic_slice_in_dim(k3, s * (qb_per // QB), qb_per // QB, 0),
        )
        idx_flat = top_idx.transpose(0, 2, 1).reshape(1, qb_per * K)
        rows.append(_gather_rows(emb_pad, idx_flat))

    out = jnp.concatenate(rows, axis=0)
    return out.reshape(batch, seq, K, dim)
